# Initial kernel scaffold; baseline (speedup 1.0000x reference)
import jax
import jax.numpy as jnp
from jax.experimental import pallas as pl


def kernel(x, edge_index, Wl0, bl0, Wr0, Wl1, bl1, Wr1, Wl2, bl2, Wr2):
    raise NotImplementedError("write your pallas kernel here")


# speedup vs baseline: 117.8863x; 117.8863x over previous
import functools

import jax
import jax.numpy as jnp
from jax import lax
from jax.experimental import pallas as pl
from jax.experimental.pallas import tpu as pltpu
from jax.experimental.pallas import tpu_sc as plsc

N = 10000
E = 320000
D = 128
NC = 2
NS = 16
NW = NC * NS
CH = 80
EPT = E // NW
NCHUNK = EPT // CH
RPT = N // NS


def _sc_agg_body(h_hbm, src_hbm, dst_hbm, z2_hbm,
                 out_hbm, cnt0_hbm, cnt1_hbm,
                 sidx, didx, rows, ones_v, stage, acc_sh, cnt_sh, gsem):
    c = lax.axis_index("c")
    s = lax.axis_index("s")
    wid = c * NS + s

    @pl.when(s < 10)
    def _():
        pltpu.sync_copy(z2_hbm.at[pl.ds(s * 1000, 1000)],
                        acc_sh.at[pl.ds(s * 1000, 1000)])
    @pl.when(s < 5)
    def _():
        for j in range(2000 // 16):
            stage[pl.ds(j * 16, 16)] = jnp.zeros((16,), jnp.float32)
        pltpu.sync_copy(stage, cnt_sh.at[pl.ds(s * 2000, 2000)])
    for j in range(CH // 16):
        ones_v[pl.ds(j * 16, 16)] = jnp.ones((16,), jnp.float32)
    plsc.subcore_barrier()

    pltpu.sync_copy(src_hbm.at[wid], sidx)
    pltpu.sync_copy(dst_hbm.at[wid], didx)

    def chunk(i, carry):
        pltpu.async_copy(h_hbm.at[sidx.at[i]], rows, gsem).wait()
        pltpu.sync_copy(rows, acc_sh.at[didx.at[i]], add=True)
        pltpu.sync_copy(ones_v, cnt_sh.at[didx.at[i]], add=True)
        return carry

    lax.fori_loop(0, NCHUNK, chunk, 0)
    plsc.subcore_barrier()

    @pl.when(s < 10)
    def _():
        pltpu.sync_copy(acc_sh.at[pl.ds(s * 1000, 1000)],
                        out_hbm.at[c, pl.ds(s * 1000, 1000)])
    @pl.when(s < 5)
    def _():
        pltpu.sync_copy(cnt_sh.at[pl.ds(s * 2000, 2000)], stage)
        @pl.when(c == 0)
        def _():
            pltpu.sync_copy(stage, cnt0_hbm.at[pl.ds(s * 2000, 2000)])
        @pl.when(c == 1)
        def _():
            pltpu.sync_copy(stage, cnt1_hbm.at[pl.ds(s * 2000, 2000)])


@functools.lru_cache(maxsize=None)
def _make_sc_agg():
    mesh = plsc.VectorSubcoreMesh(core_axis_name="c", subcore_axis_name="s")
    return pl.kernel(
        _sc_agg_body,
        out_type=[
            jax.ShapeDtypeStruct((NC, N, D), jnp.float32),
            jax.ShapeDtypeStruct((N,), jnp.float32),
            jax.ShapeDtypeStruct((N,), jnp.float32),
        ],
        mesh=mesh,
        scratch_types=[
            pltpu.VMEM((NCHUNK, CH), jnp.int32),
            pltpu.VMEM((NCHUNK, CH), jnp.int32),
            pltpu.VMEM((CH, D), jnp.float32),
            pltpu.VMEM((CH,), jnp.float32),
            pltpu.VMEM((2000,), jnp.float32),
            pltpu.VMEM_SHARED((N, D), jnp.float32),
            pltpu.VMEM_SHARED((N,), jnp.float32),
            pltpu.SemaphoreType.DMA,
        ],
        name="sage_sc_agg",
    )


BT = 2000


def _I0(*_):
    return jnp.int32(0)


def _tc_layer_body(s_ref, c0_ref, c1_ref, h_ref, wl_ref, wr_ref, b_ref,
                   out_ref):
    inv = 1.0 / jnp.maximum(c0_ref[...] + c1_ref[...], 1.0)
    mean = (s_ref[0] + s_ref[1]) * inv
    out_ref[...] = (
        jnp.dot(mean, wl_ref[...], preferred_element_type=jnp.float32)
        + jnp.dot(h_ref[...], wr_ref[...], preferred_element_type=jnp.float32)
        + b_ref[...])


@functools.lru_cache(maxsize=None)
def _make_tc_layer():
    return pl.pallas_call(
        _tc_layer_body,
        grid=(N // BT,),
        in_specs=[
            pl.BlockSpec((NC, BT, D), lambda i: (_I0(), i, _I0())),
            pl.BlockSpec((BT, 1), lambda i: (i, _I0())),
            pl.BlockSpec((BT, 1), lambda i: (i, _I0())),
            pl.BlockSpec((BT, D), lambda i: (i, _I0())),
            pl.BlockSpec((D, D), lambda i: (_I0(), _I0())),
            pl.BlockSpec((D, D), lambda i: (_I0(), _I0())),
            pl.BlockSpec((1, D), lambda i: (_I0(), _I0())),
        ],
        out_specs=pl.BlockSpec((BT, D), lambda i: (i, _I0())),
        out_shape=jax.ShapeDtypeStruct((N, D), jnp.float32),
        name="sage_tc_layer",
    )


def kernel(x, edge_index, Wl0, bl0, Wr0, Wl1, bl1, Wr1, Wl2, bl2, Wr2):
    x = x.astype(jnp.float32)
    ei = edge_index.astype(jnp.int32)
    src3 = ei[0].reshape(NW, NCHUNK, CH)
    dst3 = ei[1].reshape(NW, NCHUNK, CH)
    z2 = jnp.zeros((N, D), jnp.float32)

    agg = _make_sc_agg()
    tc = _make_tc_layer()

    params = [(Wl0, bl0, Wr0), (Wl1, bl1, Wr1), (Wl2, bl2, Wr2)]
    h = x
    outs = [x]
    c0 = c1 = None
    for (Wl, bl, Wr) in params:
        part, cp0, cp1 = agg(h, src3, dst3, z2)
        if c0 is None:
            c0 = cp0.reshape(N, 1)
            c1 = cp1.reshape(N, 1)
        h = tc(part, c0, c1,
               h,
               Wl.T.astype(jnp.float32),
               Wr.T.astype(jnp.float32),
               bl.reshape(1, D).astype(jnp.float32))
        outs.append(h)
    return jnp.concatenate(outs, axis=-1).astype(jnp.float64)


# speedup vs baseline: 146.0932x; 1.2393x over previous
import functools

import jax
import jax.numpy as jnp
from jax import lax
from jax.experimental import pallas as pl
from jax.experimental.pallas import tpu as pltpu
from jax.experimental.pallas import tpu_sc as plsc

N = 10000
E = 320000
D = 128
NC = 2
NS = 16
NW = NC * NS
CH = 80
EPT = E // NW
NCHUNK = EPT // CH
RPT = N // NS


NBUF = 2
NCHUNKP = NCHUNK + 1
NGRP = NCHUNKP // NBUF


def _sc_agg_body_counts(h_hbm, src_hbm, dst_hbm, z2_hbm,
                        out_hbm, cnt0_hbm, cnt1_hbm,
                        sidx2, didx2, rows, ones_v, stage, acc_sh, cnt_sh,
                        gsems, ssems, csems, isems):
    _sc_agg_common(h_hbm, src_hbm, dst_hbm, z2_hbm, out_hbm,
                   (cnt0_hbm, cnt1_hbm),
                   sidx2, didx2, rows, ones_v, stage, acc_sh, cnt_sh,
                   gsems, ssems, csems, isems)


def _sc_agg_body_plain(h_hbm, src_hbm, dst_hbm, z2_hbm,
                       out_hbm,
                       sidx2, didx2, rows, acc_sh, gsems, ssems, isems):
    _sc_agg_common(h_hbm, src_hbm, dst_hbm, z2_hbm, out_hbm, None,
                   sidx2, didx2, rows, None, None, acc_sh=acc_sh,
                   cnt_sh=None, gsems=gsems, ssems=ssems, csems=None,
                   isems=isems)


def _sc_agg_common(h_hbm, src_hbm, dst_hbm, z2_hbm, out_hbm, cnt_out,
                   sidx2, didx2, rows, ones_v, stage, acc_sh, cnt_sh,
                   gsems, ssems, csems, isems):
    c = lax.axis_index("c")
    s = lax.axis_index("s")
    wid = c * NS + s
    with_counts = cnt_out is not None

    @pl.when(s < 10)
    def _():
        pltpu.sync_copy(z2_hbm.at[pl.ds(s * 1000, 1000)],
                        acc_sh.at[pl.ds(s * 1000, 1000)])
    if with_counts:
        @pl.when(s < 5)
        def _():
            for j in range(2000 // 16):
                stage[pl.ds(j * 16, 16)] = jnp.zeros((16,), jnp.float32)
            pltpu.sync_copy(stage, cnt_sh.at[pl.ds(s * 2000, 2000)])
        for j in range(CH // 16):
            ones_v[pl.ds(j * 16, 16)] = jnp.ones((16,), jnp.float32)
    plsc.subcore_barrier()

    def idx_load_start(g, p):
        p = jnp.int32(p)
        pltpu.async_copy(src_hbm.at[wid, pl.ds(g * NBUF, NBUF)],
                         sidx2.at[p], isems.at[p])
        pltpu.async_copy(dst_hbm.at[wid, pl.ds(g * NBUF, NBUF)],
                         didx2.at[p], isems.at[p])

    def idx_wait(p):
        p = jnp.int32(p)
        pltpu.make_async_copy(src_hbm.at[wid, pl.ds(0, NBUF)],
                              sidx2.at[p], isems.at[p]).wait()
        pltpu.make_async_copy(dst_hbm.at[wid, pl.ds(0, NBUF)],
                              didx2.at[p], isems.at[p]).wait()

    def gather_start(p, b):
        pltpu.async_copy(h_hbm.at[sidx2.at[jnp.int32(p), jnp.int32(b)]],
                         rows.at[jnp.int32(b)], gsems.at[jnp.int32(b)])

    def gather_wait(p, b):
        pltpu.make_async_copy(
            h_hbm.at[sidx2.at[jnp.int32(p), jnp.int32(b)]],
            rows.at[jnp.int32(b)], gsems.at[jnp.int32(b)]).wait()

    def scatter_start(p, b):
        pltpu.async_copy(rows.at[jnp.int32(b)],
                         acc_sh.at[didx2.at[jnp.int32(p), jnp.int32(b)]],
                         ssems.at[jnp.int32(b)], add=True)
        if with_counts:
            pltpu.async_copy(ones_v,
                             cnt_sh.at[didx2.at[jnp.int32(p), jnp.int32(b)]],
                             csems.at[jnp.int32(b)], add=True)

    def scatter_wait(p, b):
        pltpu.make_async_copy(
            rows.at[jnp.int32(b)],
            acc_sh.at[didx2.at[jnp.int32(p), jnp.int32(b)]],
            ssems.at[jnp.int32(b)]).wait()
        if with_counts:
            pltpu.make_async_copy(
                ones_v, cnt_sh.at[didx2.at[jnp.int32(p), jnp.int32(b)]],
                csems.at[jnp.int32(b)]).wait()

    idx_load_start(jnp.int32(0), 0)
    idx_wait(0)
    idx_load_start(jnp.int32(1), 1)
    for b in range(NBUF):
        gather_start(0, b)

    def group(g, carry):
        p = lax.rem(g, jnp.int32(2))
        for b in range(NBUF):
            i = g * NBUF + b
            @pl.when(i < NCHUNK)
            def _():
                gather_wait(p, b)
                scatter_start(p, b)
        @pl.when(g + 1 < NGRP)
        def _():
            idx_wait(1 - p)
        for b in range(NBUF):
            i = g * NBUF + b
            nxt = i + NBUF
            @pl.when(i < NCHUNK)
            def _():
                scatter_wait(p, b)
            @pl.when(nxt < NCHUNK)
            def _():
                gather_start(1 - p, b)
        @pl.when(g + 2 < NGRP)
        def _():
            idx_load_start(g + 2, p)
        return carry

    lax.fori_loop(jnp.int32(0), jnp.int32(NGRP), group, 0)
    plsc.subcore_barrier()

    @pl.when(s < 10)
    def _():
        pltpu.sync_copy(acc_sh.at[pl.ds(s * 1000, 1000)],
                        out_hbm.at[c, pl.ds(s * 1000, 1000)])
    if with_counts:
        cnt0_hbm, cnt1_hbm = cnt_out
        @pl.when(s < 5)
        def _():
            pltpu.sync_copy(cnt_sh.at[pl.ds(s * 2000, 2000)], stage)
            @pl.when(c == 0)
            def _():
                pltpu.sync_copy(stage, cnt0_hbm.at[pl.ds(s * 2000, 2000)])
            @pl.when(c == 1)
            def _():
                pltpu.sync_copy(stage, cnt1_hbm.at[pl.ds(s * 2000, 2000)])


@functools.lru_cache(maxsize=None)
def _make_sc_agg(with_counts):
    mesh = plsc.VectorSubcoreMesh(core_axis_name="c", subcore_axis_name="s")
    if with_counts:
        out_type = [
            jax.ShapeDtypeStruct((NC, N, D), jnp.float32),
            jax.ShapeDtypeStruct((N,), jnp.float32),
            jax.ShapeDtypeStruct((N,), jnp.float32),
        ]
        scratch = [
            pltpu.VMEM((2, NBUF, CH), jnp.int32),
            pltpu.VMEM((2, NBUF, CH), jnp.int32),
            pltpu.VMEM((NBUF, CH, D), jnp.float32),
            pltpu.VMEM((CH,), jnp.float32),
            pltpu.VMEM((2000,), jnp.float32),
            pltpu.VMEM_SHARED((N, D), jnp.float32),
            pltpu.VMEM_SHARED((N,), jnp.float32),
            pltpu.SemaphoreType.DMA((NBUF,)),
            pltpu.SemaphoreType.DMA((NBUF,)),
            pltpu.SemaphoreType.DMA((NBUF,)),
            pltpu.SemaphoreType.DMA((2,)),
        ]
        body = _sc_agg_body_counts
    else:
        out_type = [jax.ShapeDtypeStruct((NC, N, D), jnp.float32)]
        scratch = [
            pltpu.VMEM((2, NBUF, CH), jnp.int32),
            pltpu.VMEM((2, NBUF, CH), jnp.int32),
            pltpu.VMEM((NBUF, CH, D), jnp.float32),
            pltpu.VMEM_SHARED((N, D), jnp.float32),
            pltpu.SemaphoreType.DMA((NBUF,)),
            pltpu.SemaphoreType.DMA((NBUF,)),
            pltpu.SemaphoreType.DMA((2,)),
        ]
        body = _sc_agg_body_plain
    return pl.kernel(
        body,
        out_type=out_type,
        mesh=mesh,
        scratch_types=scratch,
        name="sage_sc_agg" + ("_c" if with_counts else ""),
    )


BT = 2000


def _I0(*_):
    return jnp.int32(0)


def _tc_layer_body(s_ref, c0_ref, c1_ref, h_ref, wl_ref, wr_ref, b_ref,
                   out_ref):
    inv = 1.0 / jnp.maximum(c0_ref[...] + c1_ref[...], 1.0)
    mean = (s_ref[0] + s_ref[1]) * inv
    out_ref[...] = (
        jnp.dot(mean, wl_ref[...], preferred_element_type=jnp.float32)
        + jnp.dot(h_ref[...], wr_ref[...], preferred_element_type=jnp.float32)
        + b_ref[...])


@functools.lru_cache(maxsize=None)
def _make_tc_layer():
    return pl.pallas_call(
        _tc_layer_body,
        grid=(N // BT,),
        in_specs=[
            pl.BlockSpec((NC, BT, D), lambda i: (_I0(), i, _I0())),
            pl.BlockSpec((BT, 1), lambda i: (i, _I0())),
            pl.BlockSpec((BT, 1), lambda i: (i, _I0())),
            pl.BlockSpec((BT, D), lambda i: (i, _I0())),
            pl.BlockSpec((D, D), lambda i: (_I0(), _I0())),
            pl.BlockSpec((D, D), lambda i: (_I0(), _I0())),
            pl.BlockSpec((1, D), lambda i: (_I0(), _I0())),
        ],
        out_specs=pl.BlockSpec((BT, D), lambda i: (i, _I0())),
        out_shape=jax.ShapeDtypeStruct((N, D), jnp.float32),
        name="sage_tc_layer",
    )


def kernel(x, edge_index, Wl0, bl0, Wr0, Wl1, bl1, Wr1, Wl2, bl2, Wr2):
    x = x.astype(jnp.float32)
    ei = edge_index.astype(jnp.int32)
    src3 = jnp.pad(ei[0].reshape(NW, EPT),
                   ((0, 0), (0, CH))).reshape(NW, NCHUNKP, CH)
    dst3 = jnp.pad(ei[1].reshape(NW, EPT),
                   ((0, 0), (0, CH))).reshape(NW, NCHUNKP, CH)
    z2 = jnp.zeros((N, D), jnp.float32)

    agg_c = _make_sc_agg(True)
    agg_p = _make_sc_agg(False)
    tc = _make_tc_layer()

    params = [(Wl0, bl0, Wr0), (Wl1, bl1, Wr1), (Wl2, bl2, Wr2)]
    h = x
    outs = [x]
    c0 = c1 = None
    for (Wl, bl, Wr) in params:
        if c0 is None:
            part, cp0, cp1 = agg_c(h, src3, dst3, z2)
            c0 = cp0.reshape(N, 1)
            c1 = cp1.reshape(N, 1)
        else:
            part = agg_p(h, src3, dst3, z2)
            if isinstance(part, (list, tuple)):
                part = part[0]
        h = tc(part, c0, c1,
               h,
               Wl.T.astype(jnp.float32),
               Wr.T.astype(jnp.float32),
               bl.reshape(1, D).astype(jnp.float32))
        outs.append(h)
    return jnp.concatenate(outs, axis=-1).astype(jnp.float64)
